# baseline (speedup 1.0000x reference)
import functools

import jax
import jax.numpy as jnp
from jax import lax
from jax.experimental import pallas as pl
from jax.experimental.pallas import tpu as pltpu
from jax.experimental.pallas import tpu_sc as plsc

B = 32
C = 64
P = 512
K = 1024
N_TOK = B * P
L = 16


BB = 8


def _vq_tc_body(z_ref, cb_ref, zq_ref, idx_ref, loss_ref, ppl_ref,
                loss_acc, cnt_acc):
    b = pl.program_id(0)
    nb = pl.num_programs(0)

    @pl.when(b == 0)
    def _init():
        loss_acc[...] = jnp.zeros_like(loss_acc)
        cnt_acc[...] = jnp.zeros_like(cnt_acc)

    cb = cb_ref[...]
    cn = jnp.sum(cb * cb, axis=1, keepdims=True)
    hi = cb.astype(jnp.bfloat16)
    lo = (cb - hi.astype(jnp.float32)).astype(jnp.bfloat16)
    hilo = jnp.concatenate([hi, lo], axis=1)
    kio = lax.broadcasted_iota(jnp.int32, (K, P), 0)
    kiof = kio.astype(jnp.float32)

    ohs = None

    for j in range(BB):
        zb = z_ref[j]

        scores2 = lax.dot_general(cb, zb + zb, (((1,), (0,)), ((), ())),
                                  preferred_element_type=jnp.float32)
        rn = jnp.sum(zb * zb, axis=0, keepdims=True)
        d2 = (rn - scores2) + cn

        m = jnp.min(d2, axis=0, keepdims=True)
        idx = jnp.min(jnp.where(d2 == m, kiof, jnp.float32(K)),
                      axis=0).astype(jnp.int32)
        idx_ref[j, 0] = idx

        oh = (kio == idx[None, :]).astype(jnp.bfloat16)
        loss_acc[...] += jnp.sum(m, keepdims=True)
        ohs = oh if ohs is None else ohs + oh

        hl = lax.dot_general(hilo, oh, (((0,), (0,)), ((), ())),
                             preferred_element_type=jnp.float32)
        zqT = hl[:C] + hl[C:]
        zq_ref[j] = zb + (zqT - zb)

    cnt_acc[...] += jnp.sum(ohs, axis=1, keepdims=True,
                            dtype=jnp.float32)

    @pl.when(b == nb - 1)
    def _fini():
        loss_ref[...] = loss_acc[...] * (0.25 / (N_TOK * C))
        p = cnt_acc[...] * (1.0 / N_TOK)
        ent = jnp.sum(p * jnp.log(p + 1e-10), keepdims=True)
        ppl_ref[...] = jnp.exp(-ent)


def _tc_stage(zr, codebook):
    return pl.pallas_call(
        _vq_tc_body,
        grid=(B // BB,),
        in_specs=[
            pl.BlockSpec((BB, C, P), lambda b: (b, 0, 0)),
            pl.BlockSpec((K, C), lambda b: (0, 0)),
        ],
        out_specs=[
            pl.BlockSpec((BB, C, P), lambda b: (b, 0, 0)),
            pl.BlockSpec((BB, 1, P), lambda b: (b, 0, 0)),
            pl.BlockSpec((1, 1), lambda b: (0, 0)),
            pl.BlockSpec((1, 1), lambda b: (0, 0)),
        ],
        out_shape=[
            jax.ShapeDtypeStruct((B, C, P), jnp.float32),
            jax.ShapeDtypeStruct((B, 1, P), jnp.int32),
            jax.ShapeDtypeStruct((1, 1), jnp.float32),
            jax.ShapeDtypeStruct((1, 1), jnp.float32),
        ],
        scratch_shapes=[
            pltpu.VMEM((1, 1), jnp.float32),
            pltpu.VMEM((K, 1), jnp.float32),
        ],
    )(zr, codebook)


def _make_sc_gather():
    info = plsc.get_sparse_core_info()
    nc, ns = info.num_cores, info.num_subcores
    mesh = plsc.VectorSubcoreMesh(core_axis_name="c", subcore_axis_name="s")

    @functools.partial(
        pl.kernel, mesh=mesh,
        compiler_params=pltpu.CompilerParams(needs_layout_passes=False),
        out_type=jax.ShapeDtypeStruct((B, C * P), jnp.float32),
        scratch_types=[
            pltpu.VMEM((K * C,), jnp.float32),
            pltpu.VMEM((P,), jnp.int32),
            pltpu.VMEM((C * P,), jnp.float32),
        ],
    )
    def sc_gather(cb_hbm, idx_hbm, out_hbm, cb_v, idx_v, rows_v):
        wid = lax.axis_index("s") * nc + lax.axis_index("c")
        pltpu.sync_copy(idx_hbm.at[wid], idx_v)

        def body_g(g, _):
            base = idx_v[pl.ds(g * L, L)] * C
            off = g * L
            for c in range(0):
                vals = plsc.load_gather(cb_v, [base + c])
                rows_v[pl.ds(c * P + off, L)] = vals
            return 0

        lax.fori_loop(0, P // L, body_g, 0)

    return sc_gather


def kernel(z, codebook):
    zr = z.reshape(B, C, P)
    zq, idx, loss, ppl = _tc_stage(zr, codebook)
    z_q = zq.reshape(B, C, 8, 8, 8)
    indices = idx.reshape(B, 8, 8, 8)
    return z_q, loss.reshape(()), indices, ppl.reshape(())


# speedup vs baseline: 1.0258x; 1.0258x over previous
import jax
import jax.numpy as jnp
from jax import lax
from jax.experimental import pallas as pl
from jax.experimental.pallas import tpu as pltpu

B = 32
C = 64
P = 512
K = 1024
N_TOK = B * P
BB = 8


def _vq_tc_body(z_ref, cb_ref, zq_ref, idx_ref, loss_ref, ppl_ref,
                loss_acc, cnt_acc):
    b = pl.program_id(0)
    nb = pl.num_programs(0)

    @pl.when(b == 0)
    def _init():
        loss_acc[...] = jnp.zeros_like(loss_acc)
        cnt_acc[...] = jnp.zeros_like(cnt_acc)

    cb = cb_ref[...]
    cn = jnp.sum(cb * cb, axis=1, keepdims=True)
    hi = cb.astype(jnp.bfloat16)
    lo = (cb - hi.astype(jnp.float32)).astype(jnp.bfloat16)
    hilo = jnp.concatenate([hi, lo], axis=1)
    kio = lax.broadcasted_iota(jnp.int32, (K, P), 0)
    kiof = kio.astype(jnp.float32)

    for j in range(BB):
        zb = z_ref[j]

        scores2 = lax.dot_general(cb, zb + zb, (((1,), (0,)), ((), ())),
                                  preferred_element_type=jnp.float32)
        rn = jnp.sum(zb * zb, axis=0, keepdims=True)
        d2 = (rn - scores2) + cn

        m = jnp.min(d2, axis=0, keepdims=True)
        idx = jnp.min(jnp.where(d2 == m, kiof, jnp.float32(K)),
                      axis=0).astype(jnp.int32)
        idx_ref[j, 0] = idx

        oh = (kio == idx[None, :]).astype(jnp.bfloat16)
        loss_acc[...] += jnp.sum(m, keepdims=True)
        cnt_acc[...] += jnp.sum(oh, axis=1, keepdims=True,
                                dtype=jnp.float32)

        hl = lax.dot_general(hilo, oh, (((0,), (0,)), ((), ())),
                             preferred_element_type=jnp.float32)
        zqT = hl[:C] + hl[C:]
        zq_ref[j] = zb + (zqT - zb)

    @pl.when(b == nb - 1)
    def _fini():
        loss_ref[...] = loss_acc[...] * (0.25 / (N_TOK * C))
        p = cnt_acc[...] * (1.0 / N_TOK)
        ent = jnp.sum(p * jnp.log(p + 1e-10), keepdims=True)
        ppl_ref[...] = jnp.exp(-ent)


def _tc_stage(zr, codebook):
    return pl.pallas_call(
        _vq_tc_body,
        grid=(B // BB,),
        in_specs=[
            pl.BlockSpec((BB, C, P), lambda b: (b, 0, 0)),
            pl.BlockSpec((K, C), lambda b: (0, 0)),
        ],
        out_specs=[
            pl.BlockSpec((BB, C, P), lambda b: (b, 0, 0)),
            pl.BlockSpec((BB, 1, P), lambda b: (b, 0, 0)),
            pl.BlockSpec((1, 1), lambda b: (0, 0)),
            pl.BlockSpec((1, 1), lambda b: (0, 0)),
        ],
        out_shape=[
            jax.ShapeDtypeStruct((B, C, P), jnp.float32),
            jax.ShapeDtypeStruct((B, 1, P), jnp.int32),
            jax.ShapeDtypeStruct((1, 1), jnp.float32),
            jax.ShapeDtypeStruct((1, 1), jnp.float32),
        ],
        scratch_shapes=[
            pltpu.VMEM((1, 1), jnp.float32),
            pltpu.VMEM((K, 1), jnp.float32),
        ],
    )(zr, codebook)


def kernel(z, codebook):
    zr = z.reshape(B, C, P)
    zq, idx, loss, ppl = _tc_stage(zr, codebook)
    z_q = zq.reshape(B, C, 8, 8, 8)
    indices = idx.reshape(B, 8, 8, 8)
    return z_q, loss.reshape(()), indices, ppl.reshape(())
